# baseline (speedup 1.0000x reference)
import functools

import jax
import jax.numpy as jnp
from jax import lax
from jax.experimental import pallas as pl
from jax.experimental.pallas import tpu as pltpu
from jax.experimental.pallas import tpu_sc as plsc

NC = 2
NS = 16
NW = NC * NS


def _sc_mesh():
    return plsc.VectorSubcoreMesh(
        core_axis_name="c", subcore_axis_name="s",
        num_cores=NC, num_subcores=NS)


_SC_PARAMS = pltpu.CompilerParams(
    needs_layout_passes=False, use_tc_tiling_on_sc=False)


def _make_deg(E, N, RB):
    EPT = E // NW
    assert EPT % 16 == 0 and N % 16 == 0 and N % RB == 0

    @functools.partial(
        pl.kernel, mesh=_sc_mesh(),
        out_type=jax.ShapeDtypeStruct((N // RB, NW, RB), jnp.float32),
        compiler_params=_SC_PARAMS,
        scratch_types=[
            pltpu.VMEM((N,), jnp.float32),
            pltpu.VMEM((EPT,), jnp.int32),
            pltpu.VMEM((EPT,), jnp.float32),
        ],
    )
    def deg_kernel(src_hbm, ew_hbm, out_hbm, acc_v, src_all, ew_all):
        cid = lax.axis_index("c")
        sid = lax.axis_index("s")
        wid = sid * NC + cid
        base = wid * EPT
        pltpu.sync_copy(src_hbm.at[pl.ds(base, EPT)], src_all)
        pltpu.sync_copy(ew_hbm.at[pl.ds(base, EPT)], ew_all)
        zero = jnp.zeros((16,), jnp.float32)

        def zbody(i, carry):
            acc_v[pl.ds(i * 16, 16)] = zero
            return carry
        lax.fori_loop(0, N // 16, zbody, 0)

        def grp(g, c2):
            idx = src_all[pl.ds(g * 16, 16)]
            w = ew_all[pl.ds(g * 16, 16)]
            plsc.addupdate_scatter(acc_v, [idx], w)
            return c2
        lax.fori_loop(0, EPT // 16, grp, 0)

        for q in range(N // RB):
            pltpu.sync_copy(acc_v.at[pl.ds(q * RB, RB)], out_hbm.at[q, wid])

    return deg_kernel


def _make_cheb(E, N, D):
    H = D // 2
    EPT = E // NS
    B = 80
    NB = EPT // B
    U = 5
    S = 3
    G = 2
    RPS = N // NS
    CH = 125
    ZQ, ZREM = divmod(RPS, B)
    assert EPT % B == 0 and B % 16 == 0 and NB % U == 0
    assert H % 16 == 0 and RPS % CH == 0

    @functools.partial(
        pl.kernel, mesh=_sc_mesh(),
        out_type=(
            jax.ShapeDtypeStruct((NC * N, H), jnp.float32),
            jax.ShapeDtypeStruct((NC * N, H), jnp.float32),
            jax.ShapeDtypeStruct((NC * N, H), jnp.float32),
        ),
        compiler_params=_SC_PARAMS,
        scratch_types=[
            pltpu.VMEM_SHARED((N, H), jnp.float32),
            pltpu.VMEM((U, B, H), jnp.float32),
            pltpu.VMEM((U, B), jnp.int32),
            pltpu.VMEM((U, B), jnp.int32),
            pltpu.VMEM((U, B), jnp.float32),
            pltpu.VMEM((CH, H), jnp.float32),
            pltpu.VMEM((CH, H), jnp.float32),
            pltpu.VMEM((CH, H), jnp.float32),
            pltpu.VMEM((CH, H), jnp.float32),
            pltpu.SemaphoreType.DMA((U,)),
            pltpu.SemaphoreType.DMA((U,)),
            pltpu.SemaphoreType.DMA((U,)),
            pltpu.SemaphoreType.DMA((U,)),
            pltpu.SemaphoreType.DMA((U,)),
        ],
    )
    def cheb_kernel(u1_hbm, src_hbm, dst_hbm, ew_hbm, dis_hbm, h1_hbm,
                    tx1_hbm, tx2_hbm, u2_hbm,
                    acc_sh, rows, sidx, didx, ewr,
                    ybuf, dbuf, obuf, obuf2,
                    gsem, xsem, dsem, wsem, ssem):
        cid = lax.axis_index("c")
        sid = lax.axis_index("s")
        base = sid * EPT
        nbase = cid * N
        r0 = sid * RPS
        zero = jnp.zeros((16,), jnp.float32)

        def zero_acc():
            def zrow(i, carry):
                for c in range(H // 16):
                    rows[0, i, pl.ds(c * 16, 16)] = zero
                return carry
            lax.fori_loop(0, B, zrow, 0)
            for q in range(ZQ):
                pltpu.sync_copy(rows.at[0], acc_sh.at[pl.ds(r0 + q * B, B)])
            if ZREM:
                pltpu.sync_copy(rows.at[0, pl.ds(0, ZREM)],
                                acc_sh.at[pl.ds(r0 + ZQ * B, ZREM)])

        def scale_16(a, w16, row0):
            for j in range(16):
                w = w16[j]
                row = row0 + j
                for c in range(H // 16):
                    rows[a, row, pl.ds(c * 16, 16)] = (
                        rows[a, row, pl.ds(c * 16, 16)] * w)

        def run_pass(u_hbm):
            def issue_small(blk, slot):
                pltpu.async_copy(src_hbm.at[pl.ds(base + blk * B, B)],
                                 sidx.at[slot], xsem.at[slot])
                pltpu.async_copy(dst_hbm.at[pl.ds(base + blk * B, B)],
                                 didx.at[slot], dsem.at[slot])
                pltpu.async_copy(ew_hbm.at[pl.ds(base + blk * B, B)],
                                 ewr.at[slot], wsem.at[slot])

            def issue_gather(blk, slot):
                pltpu.make_async_copy(src_hbm.at[pl.ds(base + blk * B, B)],
                                      sidx.at[slot], xsem.at[slot]).wait()
                for g in range(B // 16):
                    v = sidx[slot, pl.ds(g * 16, 16)]
                    sidx[slot, pl.ds(g * 16, 16)] = v + nbase
                pltpu.async_copy(u_hbm.at[sidx.at[slot]],
                                 rows.at[slot], gsem.at[slot])

            def wait_gather(blk, slot):
                pltpu.make_async_copy(u_hbm.at[sidx.at[slot]],
                                      rows.at[slot], gsem.at[slot]).wait()
                pltpu.make_async_copy(ew_hbm.at[pl.ds(base + blk * B, B)],
                                      ewr.at[slot], wsem.at[slot]).wait()

            def wait_scatter(slot):
                pltpu.make_async_copy(rows.at[slot], acc_sh.at[didx.at[slot]],
                                      ssem.at[slot]).wait()

            for a in range(S):
                issue_small(a, a)
            for a in range(G):
                issue_gather(a, a)

            def step(b, a):
                slot_s = (a + S) % U
                slot_g = (a + G) % U

                @pl.when((b >= U - S) & (b + S < NB))
                def _():
                    wait_scatter(slot_s)

                @pl.when(b + S < NB)
                def _():
                    issue_small(b + S, slot_s)

                @pl.when(b + G < NB)
                def _():
                    issue_gather(b + G, slot_g)

                wait_gather(b, a)

                def scale(g, c2):
                    scale_16(a, ewr[a, pl.ds(g * 16, 16)], g * 16)
                    return c2
                lax.fori_loop(0, B // 16, scale, 0)

                pltpu.make_async_copy(dst_hbm.at[pl.ds(base + b * B, B)],
                                      didx.at[a], dsem.at[a]).wait()
                pltpu.async_copy(rows.at[a], acc_sh.at[didx.at[a]],
                                 ssem.at[a], add=True)

            def outer(ob, carry):
                for a in range(U):
                    step(ob * U + a, a)
                return carry
            lax.fori_loop(0, NB // U, outer, 0)
            for x in range(NB - U, NB):
                wait_scatter(x % U)

        zero_acc()
        plsc.subcore_barrier()
        run_pass(u1_hbm)
        plsc.subcore_barrier()

        for q in range(RPS // CH):
            rq = r0 + q * CH
            pltpu.sync_copy(acc_sh.at[pl.ds(rq, CH)], ybuf)
            pltpu.sync_copy(dis_hbm.at[pl.ds(rq, CH)], dbuf)

            def mid_row(i, carry):
                for c in range(H // 16):
                    y = ybuf[i, pl.ds(c * 16, 16)]
                    d = dbuf[i, pl.ds(c * 16, 16)]
                    t1 = -(d * y)
                    obuf[i, pl.ds(c * 16, 16)] = t1
                    obuf2[i, pl.ds(c * 16, 16)] = d * t1
                return carry
            lax.fori_loop(0, CH, mid_row, 0)
            pltpu.sync_copy(obuf, tx1_hbm.at[pl.ds(nbase + rq, CH)])
            pltpu.sync_copy(obuf2, u2_hbm.at[pl.ds(nbase + rq, CH)])
        zero_acc()
        plsc.subcore_barrier()

        run_pass(u2_hbm)
        plsc.subcore_barrier()

        for q in range(RPS // CH):
            rq = r0 + q * CH
            pltpu.sync_copy(acc_sh.at[pl.ds(rq, CH)], ybuf)
            pltpu.sync_copy(dis_hbm.at[pl.ds(rq, CH)], dbuf)
            pltpu.sync_copy(h1_hbm.at[pl.ds(nbase + rq, CH)], obuf2)

            def tail_row(i, carry):
                for c in range(H // 16):
                    y = ybuf[i, pl.ds(c * 16, 16)]
                    d = dbuf[i, pl.ds(c * 16, 16)]
                    h = obuf2[i, pl.ds(c * 16, 16)]
                    obuf[i, pl.ds(c * 16, 16)] = (-2.0 * d) * y - h
                return carry
            lax.fori_loop(0, CH, tail_row, 0)
            pltpu.sync_copy(obuf, tx2_hbm.at[pl.ds(nbase + rq, CH)])

    return cheb_kernel


def _gates(z, D):
    zi = jax.nn.sigmoid(z[:, :D])
    zf = jax.nn.sigmoid(z[:, D:2 * D])
    zc = jnp.tanh(z[:, 2 * D:3 * D])
    zo = jax.nn.sigmoid(z[:, 3 * D:])
    return zi, zf, zc, zo


def _make_dense0(N, D, RB):
    Hh = D // 2

    def body(x_ref, w_ref, b_ref, degp_ref,
             h1_ref, c1_ref, u1h_ref, h1h_ref, dish_ref):
        z = jnp.dot(x_ref[...], w_ref[...],
                    preferred_element_type=jnp.float32) + b_ref[...]
        zi, zf, zc, zo = _gates(z, D)
        c1 = zi * zc
        h1 = zo * jnp.tanh(c1)
        c1_ref[...] = c1
        h1_ref[...] = h1
        deg = jnp.sum(degp_ref[0], axis=0)
        deg_safe = jnp.where(deg > 0, deg, 1.0)
        dis = jnp.where(deg > 0, lax.rsqrt(deg_safe), 0.0)
        disb = jnp.broadcast_to(dis[:, None], (RB, D))
        u1 = disb * h1
        u1h_ref[0] = u1[:, :Hh]
        u1h_ref[1] = u1[:, Hh:]
        h1h_ref[0] = h1[:, :Hh]
        h1h_ref[1] = h1[:, Hh:]
        dish_ref[...] = disb[:, :Hh]

    return pl.pallas_call(
        body,
        grid=(N // RB,),
        in_specs=[
            pl.BlockSpec((RB, D), lambda i: (i, 0)),
            pl.BlockSpec((D, 4 * D), lambda i: (0, 0)),
            pl.BlockSpec((1, 4 * D), lambda i: (0, 0)),
            pl.BlockSpec((1, NW, RB), lambda i: (i, 0, 0)),
        ],
        out_specs=[
            pl.BlockSpec((RB, D), lambda i: (i, 0)),
            pl.BlockSpec((RB, D), lambda i: (i, 0)),
            pl.BlockSpec((NC, RB, Hh), lambda i: (0, i, 0)),
            pl.BlockSpec((NC, RB, Hh), lambda i: (0, i, 0)),
            pl.BlockSpec((RB, Hh), lambda i: (i, 0)),
        ],
        out_shape=[
            jax.ShapeDtypeStruct((N, D), jnp.float32),
            jax.ShapeDtypeStruct((N, D), jnp.float32),
            jax.ShapeDtypeStruct((NC, N, Hh), jnp.float32),
            jax.ShapeDtypeStruct((NC, N, Hh), jnp.float32),
            jax.ShapeDtypeStruct((N, Hh), jnp.float32),
        ],
    )


def _make_final(N, D, RB):
    Hh = D // 2

    def body(x_ref, h1_ref, c1_ref, tx1_ref, tx2_ref, w_ref, b_ref, out_ref):
        tx1 = jnp.concatenate([tx1_ref[0], tx1_ref[1]], axis=1)
        tx2 = jnp.concatenate([tx2_ref[0], tx2_ref[1]], axis=1)
        a = jnp.concatenate([x_ref[...], h1_ref[...], tx1, tx2], axis=1)
        z = jnp.dot(a, w_ref[...],
                    preferred_element_type=jnp.float32) + b_ref[...]
        zi, zf, zc, zo = _gates(z, D)
        c2 = zf * c1_ref[...] + zi * zc
        h2 = zo * jnp.tanh(c2)
        out_ref[...] = jnp.maximum(h2, 0.0)

    return pl.pallas_call(
        body,
        grid=(N // RB,),
        in_specs=[
            pl.BlockSpec((RB, D), lambda i: (i, 0)),
            pl.BlockSpec((RB, D), lambda i: (i, 0)),
            pl.BlockSpec((RB, D), lambda i: (i, 0)),
            pl.BlockSpec((NC, RB, Hh), lambda i: (0, i, 0)),
            pl.BlockSpec((NC, RB, Hh), lambda i: (0, i, 0)),
            pl.BlockSpec((4 * D, 4 * D), lambda i: (0, 0)),
            pl.BlockSpec((1, 4 * D), lambda i: (0, 0)),
        ],
        out_specs=pl.BlockSpec((RB, D), lambda i: (i, 0)),
        out_shape=jax.ShapeDtypeStruct((N, D), jnp.float32),
    )


def kernel(edge_index_list, node_feats_list, edge_feats_list, nodes_mask_list,
           W_i, W_f, W_c, W_o, b_i, b_f, b_c, b_o,
           conv_i_W, conv_i_b, conv_f_W, conv_f_b,
           conv_c_W, conv_c_b, conv_o_W, conv_o_b):
    del nodes_mask_list
    Tn, N, D = node_feats_list.shape
    E = edge_index_list.shape[2]
    assert Tn == 2
    Hh = D // 2

    X0 = node_feats_list[0]
    X1 = node_feats_list[1]
    src = edge_index_list[1, 0]
    dst = edge_index_list[1, 1]
    ew = edge_feats_list[1]

    Wx = jnp.concatenate([W_i, W_f, W_c, W_o], axis=1)
    Wk = [jnp.concatenate([conv_i_W[k], conv_f_W[k],
                           conv_c_W[k], conv_o_W[k]], axis=1)
          for k in range(3)]
    Wbig = jnp.concatenate([Wx, Wk[0], Wk[1], Wk[2]], axis=0)
    bias = (jnp.concatenate([b_i, b_f, b_c, b_o], axis=1)
            + jnp.concatenate([conv_i_b, conv_f_b,
                               conv_c_b, conv_o_b])[None, :])

    RB = 1000
    deg_p = _make_deg(E, N, RB)(src, ew)
    H1, C1, U1h, H1h, dish = _make_dense0(N, D, RB)(
        X0, Wx, bias, deg_p)
    U1h = U1h.reshape(NC * N, Hh)
    H1h = H1h.reshape(NC * N, Hh)
    Tx1h, Tx2h, _u2 = _make_cheb(E, N, D)(
        U1h, src, dst, ew, dish, H1h)
    Tx1h = Tx1h.reshape(NC, N, Hh)
    Tx2h = Tx2h.reshape(NC, N, Hh)
    out = _make_final(N, D, RB)(X1, H1, C1, Tx1h, Tx2h, Wbig, bias)
    return out


# speedup vs baseline: 2.1644x; 2.1644x over previous
import functools

import jax
import jax.numpy as jnp
from jax import lax
from jax.experimental import pallas as pl
from jax.experimental.pallas import tpu as pltpu
from jax.experimental.pallas import tpu_sc as plsc

NC = 2
NS = 16
NW = NC * NS


def _sc_mesh():
    return plsc.VectorSubcoreMesh(
        core_axis_name="c", subcore_axis_name="s",
        num_cores=NC, num_subcores=NS)


def _make_deg(E, N, RB):
    EPT = E // NW
    assert EPT % 16 == 0 and N % 16 == 0 and N % RB == 0

    @functools.partial(
        pl.kernel, mesh=_sc_mesh(),
        out_type=jax.ShapeDtypeStruct((N // RB, NW, RB), jnp.float32),
        compiler_params=pltpu.CompilerParams(
            needs_layout_passes=False, use_tc_tiling_on_sc=False),
        scratch_types=[
            pltpu.VMEM((N,), jnp.float32),
            pltpu.VMEM((EPT,), jnp.int32),
            pltpu.VMEM((EPT,), jnp.float32),
        ],
    )
    def deg_kernel(src_hbm, ew_hbm, out_hbm, acc_v, src_all, ew_all):
        cid = lax.axis_index("c")
        sid = lax.axis_index("s")
        wid = sid * NC + cid
        base = wid * EPT
        pltpu.sync_copy(src_hbm.at[pl.ds(base, EPT)], src_all)
        pltpu.sync_copy(ew_hbm.at[pl.ds(base, EPT)], ew_all)
        zero = jnp.zeros((16,), jnp.float32)

        def zbody(i, carry):
            acc_v[pl.ds(i * 16, 16)] = zero
            return carry
        lax.fori_loop(0, N // 16, zbody, 0)

        def grp(g, c2):
            idx = src_all[pl.ds(g * 16, 16)]
            w = ew_all[pl.ds(g * 16, 16)]
            plsc.addupdate_scatter(acc_v, [idx], w)
            return c2
        lax.fori_loop(0, EPT // 16, grp, 0)

        for q in range(N // RB):
            pltpu.sync_copy(acc_v.at[pl.ds(q * RB, RB)], out_hbm.at[q, wid])

    return deg_kernel


def _make_prop(E, N, D):
    EPT = E // NW
    B = 40
    NB = EPT // B
    U = 5
    S = 3
    G = 2
    NPEEL = NB % U
    RPS = N // NS
    ZQ, ZREM = divmod(RPS, B)
    assert EPT % B == 0 and B % 8 == 0 and D % 16 == 0

    @functools.partial(
        pl.kernel, mesh=_sc_mesh(),
        out_type=jax.ShapeDtypeStruct((NC, N, D), jnp.float32),
        compiler_params=pltpu.CompilerParams(
            needs_layout_passes=False, use_tc_tiling_on_sc=False),
        scratch_types=[
            pltpu.VMEM_SHARED((N, D), jnp.float32),
            pltpu.VMEM((U, B, D), jnp.float32),
            pltpu.VMEM((U, B), jnp.int32),
            pltpu.VMEM((U, B), jnp.int32),
            pltpu.VMEM((U, B), jnp.float32),
            pltpu.SemaphoreType.DMA((U,)),
            pltpu.SemaphoreType.DMA((U,)),
            pltpu.SemaphoreType.DMA((U,)),
            pltpu.SemaphoreType.DMA((U,)),
            pltpu.SemaphoreType.DMA((U,)),
        ],
    )
    def prop_kernel(u_hbm, src_hbm, dst_hbm, ew_hbm, out_hbm,
                    acc_sh, rows, sidx, didx, ewr,
                    gsem, xsem, dsem, wsem, ssem):
        cid = lax.axis_index("c")
        sid = lax.axis_index("s")
        wid = sid * NC + cid
        base = wid * EPT

        zero = jnp.zeros((16,), jnp.float32)

        def zrow(i, carry):
            for c in range(D // 16):
                rows[0, i, pl.ds(c * 16, 16)] = zero
            return carry
        lax.fori_loop(0, B, zrow, 0)
        for q in range(ZQ):
            pltpu.sync_copy(rows.at[0],
                            acc_sh.at[pl.ds(sid * RPS + q * B, B)])
        if ZREM:
            pltpu.sync_copy(rows.at[0, pl.ds(0, ZREM)],
                            acc_sh.at[pl.ds(sid * RPS + ZQ * B, ZREM)])
        plsc.subcore_barrier()

        def issue_small(blk, slot):
            pltpu.async_copy(src_hbm.at[pl.ds(base + blk * B, B)],
                             sidx.at[slot], xsem.at[slot])
            pltpu.async_copy(dst_hbm.at[pl.ds(base + blk * B, B)],
                             didx.at[slot], dsem.at[slot])
            pltpu.async_copy(ew_hbm.at[pl.ds(base + blk * B, B)],
                             ewr.at[slot], wsem.at[slot])

        def issue_gather(blk, slot):
            pltpu.make_async_copy(src_hbm.at[pl.ds(base + blk * B, B)],
                                  sidx.at[slot], xsem.at[slot]).wait()
            pltpu.async_copy(u_hbm.at[sidx.at[slot]],
                             rows.at[slot], gsem.at[slot])

        def wait_gather(blk, slot):
            pltpu.make_async_copy(u_hbm.at[sidx.at[slot]],
                                  rows.at[slot], gsem.at[slot]).wait()
            pltpu.make_async_copy(ew_hbm.at[pl.ds(base + blk * B, B)],
                                  ewr.at[slot], wsem.at[slot]).wait()

        def wait_scatter(slot):
            pltpu.make_async_copy(rows.at[slot], acc_sh.at[didx.at[slot]],
                                  ssem.at[slot]).wait()

        for a in range(S):
            issue_small(a, a)
        for a in range(G):
            issue_gather(a, a)

        def scale_16(a, w16, row0):
            for j in range(16):
                w = w16[j]
                row = row0 + j
                for c in range(D // 16):
                    rows[a, row, pl.ds(c * 16, 16)] = (
                        rows[a, row, pl.ds(c * 16, 16)] * w)

        def step(b, a):
            slot_s = (a + S) % U
            slot_g = (a + G) % U

            @pl.when((b >= U - S) & (b + S < NB))
            def _():
                wait_scatter(slot_s)

            @pl.when(b + S < NB)
            def _():
                issue_small(b + S, slot_s)

            @pl.when(b + G < NB)
            def _():
                issue_gather(b + G, slot_g)

            wait_gather(b, a)

            def scale(g, c2):
                scale_16(a, ewr[a, pl.ds(g * 16, 16)], g * 16)
                return c2
            lax.fori_loop(0, B // 16, scale, 0)
            if B % 16:
                rem = B % 16
                w16 = ewr[a, pl.ds(B - 16, 16)]
                for j in range(16 - rem, 16):
                    w = w16[j]
                    row = B - 16 + j
                    for c in range(D // 16):
                        rows[a, row, pl.ds(c * 16, 16)] = (
                            rows[a, row, pl.ds(c * 16, 16)] * w)

            pltpu.make_async_copy(dst_hbm.at[pl.ds(base + b * B, B)],
                                  didx.at[a], dsem.at[a]).wait()
            pltpu.async_copy(rows.at[a], acc_sh.at[didx.at[a]],
                             ssem.at[a], add=True)

        def outer(ob, carry):
            for a in range(U):
                step(ob * U + a, a)
            return carry
        lax.fori_loop(0, NB // U, outer, 0)
        for p in range(NB - NPEEL, NB):
            step(p, p % U)

        for x in range(NB - U, NB):
            wait_scatter(x % U)

        plsc.subcore_barrier()
        pltpu.sync_copy(acc_sh.at[pl.ds(sid * RPS, RPS)],
                        out_hbm.at[cid, pl.ds(sid * RPS, RPS)])

    return prop_kernel


def _gates(z, D):
    zi = jax.nn.sigmoid(z[:, :D])
    zf = jax.nn.sigmoid(z[:, D:2 * D])
    zc = jnp.tanh(z[:, 2 * D:3 * D])
    zo = jax.nn.sigmoid(z[:, 3 * D:])
    return zi, zf, zc, zo


def _make_dense0(N, D, RB):
    def body(x_ref, w_ref, b_ref, degp_ref,
             h1_ref, c1_ref, u1_ref, disb_ref):
        z = jnp.dot(x_ref[...], w_ref[...],
                    preferred_element_type=jnp.float32) + b_ref[...]
        zi, zf, zc, zo = _gates(z, D)
        c1 = zi * zc
        h1 = zo * jnp.tanh(c1)
        c1_ref[...] = c1
        h1_ref[...] = h1
        deg = jnp.sum(degp_ref[0], axis=0)
        deg_safe = jnp.where(deg > 0, deg, 1.0)
        dis = jnp.where(deg > 0, lax.rsqrt(deg_safe), 0.0)
        disb = jnp.broadcast_to(dis[:, None], (RB, D))
        disb_ref[...] = disb
        u1_ref[...] = disb * h1

    return pl.pallas_call(
        body,
        grid=(N // RB,),
        in_specs=[
            pl.BlockSpec((RB, D), lambda i: (i, 0)),
            pl.BlockSpec((D, 4 * D), lambda i: (0, 0)),
            pl.BlockSpec((1, 4 * D), lambda i: (0, 0)),
            pl.BlockSpec((1, NW, RB), lambda i: (i, 0, 0)),
        ],
        out_specs=[
            pl.BlockSpec((RB, D), lambda i: (i, 0)),
            pl.BlockSpec((RB, D), lambda i: (i, 0)),
            pl.BlockSpec((RB, D), lambda i: (i, 0)),
            pl.BlockSpec((RB, D), lambda i: (i, 0)),
        ],
        out_shape=[
            jax.ShapeDtypeStruct((N, D), jnp.float32),
            jax.ShapeDtypeStruct((N, D), jnp.float32),
            jax.ShapeDtypeStruct((N, D), jnp.float32),
            jax.ShapeDtypeStruct((N, D), jnp.float32),
        ],
    )


def _make_mid(N, D, RB):
    def body(y_ref, disb_ref, tx1_ref, u2_ref):
        ysum = y_ref[0] + y_ref[1]
        disb = disb_ref[...]
        tx1 = -disb * ysum
        tx1_ref[...] = tx1
        u2_ref[...] = disb * tx1

    return pl.pallas_call(
        body,
        grid=(N // RB,),
        in_specs=[
            pl.BlockSpec((NC, RB, D), lambda i: (0, i, 0)),
            pl.BlockSpec((RB, D), lambda i: (i, 0)),
        ],
        out_specs=[
            pl.BlockSpec((RB, D), lambda i: (i, 0)),
            pl.BlockSpec((RB, D), lambda i: (i, 0)),
        ],
        out_shape=[
            jax.ShapeDtypeStruct((N, D), jnp.float32),
            jax.ShapeDtypeStruct((N, D), jnp.float32),
        ],
    )


def _make_final(N, D, RB):
    def body(x_ref, h1_ref, c1_ref, tx1_ref, disb_ref, y_ref, w_ref, b_ref,
             out_ref):
        h1 = h1_ref[...]
        tx2 = -2.0 * disb_ref[...] * (y_ref[0] + y_ref[1]) - h1
        a = jnp.concatenate([x_ref[...], h1, tx1_ref[...], tx2], axis=1)
        z = jnp.dot(a, w_ref[...],
                    preferred_element_type=jnp.float32) + b_ref[...]
        zi, zf, zc, zo = _gates(z, D)
        c2 = zf * c1_ref[...] + zi * zc
        h2 = zo * jnp.tanh(c2)
        out_ref[...] = jnp.maximum(h2, 0.0)

    return pl.pallas_call(
        body,
        grid=(N // RB,),
        in_specs=[
            pl.BlockSpec((RB, D), lambda i: (i, 0)),
            pl.BlockSpec((RB, D), lambda i: (i, 0)),
            pl.BlockSpec((RB, D), lambda i: (i, 0)),
            pl.BlockSpec((RB, D), lambda i: (i, 0)),
            pl.BlockSpec((RB, D), lambda i: (i, 0)),
            pl.BlockSpec((NC, RB, D), lambda i: (0, i, 0)),
            pl.BlockSpec((4 * D, 4 * D), lambda i: (0, 0)),
            pl.BlockSpec((1, 4 * D), lambda i: (0, 0)),
        ],
        out_specs=pl.BlockSpec((RB, D), lambda i: (i, 0)),
        out_shape=jax.ShapeDtypeStruct((N, D), jnp.float32),
    )


def kernel(edge_index_list, node_feats_list, edge_feats_list, nodes_mask_list,
           W_i, W_f, W_c, W_o, b_i, b_f, b_c, b_o,
           conv_i_W, conv_i_b, conv_f_W, conv_f_b,
           conv_c_W, conv_c_b, conv_o_W, conv_o_b):
    del nodes_mask_list
    Tn, N, D = node_feats_list.shape
    E = edge_index_list.shape[2]
    assert Tn == 2

    X0 = node_feats_list[0]
    X1 = node_feats_list[1]
    src = edge_index_list[1, 0]
    dst = edge_index_list[1, 1]
    ew = edge_feats_list[1]

    Wx = jnp.concatenate([W_i, W_f, W_c, W_o], axis=1)
    Wk = [jnp.concatenate([conv_i_W[k], conv_f_W[k],
                           conv_c_W[k], conv_o_W[k]], axis=1)
          for k in range(3)]
    Wbig = jnp.concatenate([Wx, Wk[0], Wk[1], Wk[2]], axis=0)
    bias = (jnp.concatenate([b_i, b_f, b_c, b_o], axis=1)
            + jnp.concatenate([conv_i_b, conv_f_b,
                               conv_c_b, conv_o_b])[None, :])

    RB = 1000
    deg_p = _make_deg(E, N, RB)(src, ew)
    H1, C1, U1, disb = _make_dense0(N, D, RB)(
        X0, Wx, bias, deg_p)
    Y1 = _make_prop(E, N, D)(U1, src, dst, ew)
    Tx1, U2 = _make_mid(N, D, RB)(Y1, disb)
    Y2 = _make_prop(E, N, D)(U2, src, dst, ew)
    out = _make_final(N, D, RB)(X1, H1, C1, Tx1, disb, Y2, Wbig, bias)
    return out


# speedup vs baseline: 2.2748x; 1.0510x over previous
import functools

import jax
import jax.numpy as jnp
from jax import lax
from jax.experimental import pallas as pl
from jax.experimental.pallas import tpu as pltpu
from jax.experimental.pallas import tpu_sc as plsc

NC = 2
NS = 16
NW = NC * NS


def _sc_mesh():
    return plsc.VectorSubcoreMesh(
        core_axis_name="c", subcore_axis_name="s",
        num_cores=NC, num_subcores=NS)


def _make_deg(E, N, RB):
    EPT = E // NW
    assert EPT % 16 == 0 and N % 16 == 0 and N % RB == 0

    @functools.partial(
        pl.kernel, mesh=_sc_mesh(),
        out_type=jax.ShapeDtypeStruct((N // RB, NW, RB), jnp.float32),
        compiler_params=pltpu.CompilerParams(
            needs_layout_passes=False, use_tc_tiling_on_sc=False),
        scratch_types=[
            pltpu.VMEM((N,), jnp.float32),
            pltpu.VMEM((EPT,), jnp.int32),
            pltpu.VMEM((EPT,), jnp.float32),
        ],
    )
    def deg_kernel(ei_hbm, ef_hbm, out_hbm, acc_v, src_all, ew_all):
        cid = lax.axis_index("c")
        sid = lax.axis_index("s")
        wid = sid * NC + cid
        base = wid * EPT
        pltpu.sync_copy(ei_hbm.at[1, 0, pl.ds(base, EPT)], src_all)
        pltpu.sync_copy(ef_hbm.at[1, pl.ds(base, EPT)], ew_all)
        zero = jnp.zeros((16,), jnp.float32)

        def zbody(i, carry):
            acc_v[pl.ds(i * 16, 16)] = zero
            return carry
        lax.fori_loop(0, N // 16, zbody, 0)

        def grp(g, c2):
            idx = src_all[pl.ds(g * 16, 16)]
            w = ew_all[pl.ds(g * 16, 16)]
            plsc.addupdate_scatter(acc_v, [idx], w)
            return c2
        lax.fori_loop(0, EPT // 16, grp, 0)

        for q in range(N // RB):
            pltpu.sync_copy(acc_v.at[pl.ds(q * RB, RB)], out_hbm.at[q, wid])

    return deg_kernel


def _make_prop(E, N, D):
    EPT = E // NW
    B = 40
    NB = EPT // B
    U = 5
    S = 3
    G = 2
    NPEEL = NB % U
    RPS = N // NS
    ZQ, ZREM = divmod(RPS, B)
    assert EPT % B == 0 and B % 8 == 0 and D % 16 == 0

    @functools.partial(
        pl.kernel, mesh=_sc_mesh(),
        out_type=jax.ShapeDtypeStruct((NC, N, D), jnp.float32),
        compiler_params=pltpu.CompilerParams(
            needs_layout_passes=False, use_tc_tiling_on_sc=False),
        scratch_types=[
            pltpu.VMEM_SHARED((N, D), jnp.float32),
            pltpu.VMEM((U, B, D), jnp.float32),
            pltpu.VMEM((U, B), jnp.int32),
            pltpu.VMEM((U, B), jnp.int32),
            pltpu.VMEM((U, B), jnp.float32),
            pltpu.SemaphoreType.DMA((U,)),
            pltpu.SemaphoreType.DMA((U,)),
            pltpu.SemaphoreType.DMA((U,)),
            pltpu.SemaphoreType.DMA((U,)),
            pltpu.SemaphoreType.DMA((U,)),
        ],
    )
    def prop_kernel(u_hbm, ei_hbm, ef_hbm, out_hbm,
                    acc_sh, rows, sidx, didx, ewr,
                    gsem, xsem, dsem, wsem, ssem):
        cid = lax.axis_index("c")
        sid = lax.axis_index("s")
        wid = sid * NC + cid
        base = wid * EPT

        zero = jnp.zeros((16,), jnp.float32)

        def zrow(i, carry):
            for c in range(D // 16):
                rows[0, i, pl.ds(c * 16, 16)] = zero
            return carry
        lax.fori_loop(0, B, zrow, 0)
        for q in range(ZQ):
            pltpu.sync_copy(rows.at[0],
                            acc_sh.at[pl.ds(sid * RPS + q * B, B)])
        if ZREM:
            pltpu.sync_copy(rows.at[0, pl.ds(0, ZREM)],
                            acc_sh.at[pl.ds(sid * RPS + ZQ * B, ZREM)])
        plsc.subcore_barrier()

        def issue_small(blk, slot):
            pltpu.async_copy(ei_hbm.at[1, 0, pl.ds(base + blk * B, B)],
                             sidx.at[slot], xsem.at[slot])
            pltpu.async_copy(ei_hbm.at[1, 1, pl.ds(base + blk * B, B)],
                             didx.at[slot], dsem.at[slot])
            pltpu.async_copy(ef_hbm.at[1, pl.ds(base + blk * B, B)],
                             ewr.at[slot], wsem.at[slot])

        def issue_gather(blk, slot):
            pltpu.make_async_copy(ei_hbm.at[1, 0, pl.ds(base + blk * B, B)],
                                  sidx.at[slot], xsem.at[slot]).wait()
            pltpu.async_copy(u_hbm.at[sidx.at[slot]],
                             rows.at[slot], gsem.at[slot])

        def wait_gather(blk, slot):
            pltpu.make_async_copy(u_hbm.at[sidx.at[slot]],
                                  rows.at[slot], gsem.at[slot]).wait()
            pltpu.make_async_copy(ef_hbm.at[1, pl.ds(base + blk * B, B)],
                                  ewr.at[slot], wsem.at[slot]).wait()

        def wait_scatter(slot):
            pltpu.make_async_copy(rows.at[slot], acc_sh.at[didx.at[slot]],
                                  ssem.at[slot]).wait()

        for a in range(S):
            issue_small(a, a)
        for a in range(G):
            issue_gather(a, a)

        def scale_16(a, w16, row0):
            for j in range(16):
                w = w16[j]
                row = row0 + j
                for c in range(D // 16):
                    rows[a, row, pl.ds(c * 16, 16)] = (
                        rows[a, row, pl.ds(c * 16, 16)] * w)

        def step(b, a):
            slot_s = (a + S) % U
            slot_g = (a + G) % U

            @pl.when((b >= U - S) & (b + S < NB))
            def _():
                wait_scatter(slot_s)

            @pl.when(b + S < NB)
            def _():
                issue_small(b + S, slot_s)

            @pl.when(b + G < NB)
            def _():
                issue_gather(b + G, slot_g)

            wait_gather(b, a)

            def scale(g, c2):
                scale_16(a, ewr[a, pl.ds(g * 16, 16)], g * 16)
                return c2
            lax.fori_loop(0, B // 16, scale, 0)
            if B % 16:
                rem = B % 16
                w16 = ewr[a, pl.ds(B - 16, 16)]
                for j in range(16 - rem, 16):
                    w = w16[j]
                    row = B - 16 + j
                    for c in range(D // 16):
                        rows[a, row, pl.ds(c * 16, 16)] = (
                            rows[a, row, pl.ds(c * 16, 16)] * w)

            pltpu.make_async_copy(ei_hbm.at[1, 1, pl.ds(base + b * B, B)],
                                  didx.at[a], dsem.at[a]).wait()
            pltpu.async_copy(rows.at[a], acc_sh.at[didx.at[a]],
                             ssem.at[a], add=True)

        def outer(ob, carry):
            for a in range(U):
                step(ob * U + a, a)
            return carry
        lax.fori_loop(0, NB // U, outer, 0)
        for p in range(NB - NPEEL, NB):
            step(p, p % U)

        for x in range(NB - U, NB):
            wait_scatter(x % U)

        plsc.subcore_barrier()
        pltpu.sync_copy(acc_sh.at[pl.ds(sid * RPS, RPS)],
                        out_hbm.at[cid, pl.ds(sid * RPS, RPS)])

    return prop_kernel


def _gates(z, D):
    zi = jax.nn.sigmoid(z[:, :D])
    zf = jax.nn.sigmoid(z[:, D:2 * D])
    zc = jnp.tanh(z[:, 2 * D:3 * D])
    zo = jax.nn.sigmoid(z[:, 3 * D:])
    return zi, zf, zc, zo


def _make_dense0(N, D, RB):
    def body(x_ref, w_ref, b_ref, degp_ref,
             h1_ref, c1_ref, u1_ref, disb_ref):
        z = jnp.dot(x_ref[0].astype(jnp.bfloat16),
                    w_ref[...].astype(jnp.bfloat16),
                    preferred_element_type=jnp.float32) + b_ref[...]
        zi, zf, zc, zo = _gates(z, D)
        c1 = zi * zc
        h1 = zo * jnp.tanh(c1)
        c1_ref[...] = c1
        h1_ref[...] = h1
        deg = jnp.sum(degp_ref[0], axis=0)
        deg_safe = jnp.where(deg > 0, deg, 1.0)
        dis = jnp.where(deg > 0, lax.rsqrt(deg_safe), 0.0)
        disb = jnp.broadcast_to(dis[:, None], (RB, D))
        disb_ref[...] = disb
        u1_ref[...] = disb * h1

    return pl.pallas_call(
        body,
        grid=(N // RB,),
        in_specs=[
            pl.BlockSpec((1, RB, D), lambda i: (0, i, 0)),
            pl.BlockSpec((D, 4 * D), lambda i: (0, 0)),
            pl.BlockSpec((1, 4 * D), lambda i: (0, 0)),
            pl.BlockSpec((1, NW, RB), lambda i: (i, 0, 0)),
        ],
        out_specs=[
            pl.BlockSpec((RB, D), lambda i: (i, 0)),
            pl.BlockSpec((RB, D), lambda i: (i, 0)),
            pl.BlockSpec((RB, D), lambda i: (i, 0)),
            pl.BlockSpec((RB, D), lambda i: (i, 0)),
        ],
        out_shape=[
            jax.ShapeDtypeStruct((N, D), jnp.float32),
            jax.ShapeDtypeStruct((N, D), jnp.float32),
            jax.ShapeDtypeStruct((N, D), jnp.float32),
            jax.ShapeDtypeStruct((N, D), jnp.float32),
        ],
    )


def _make_mid(N, D, RB):
    def body(y_ref, disb_ref, u2_ref):
        disb = disb_ref[...]
        u2_ref[...] = -(disb * disb) * (y_ref[0] + y_ref[1])

    return pl.pallas_call(
        body,
        grid=(N // RB,),
        in_specs=[
            pl.BlockSpec((NC, RB, D), lambda i: (0, i, 0)),
            pl.BlockSpec((RB, D), lambda i: (i, 0)),
        ],
        out_specs=pl.BlockSpec((RB, D), lambda i: (i, 0)),
        out_shape=jax.ShapeDtypeStruct((N, D), jnp.float32),
    )


def _make_final(N, D, RB):
    def body(x_ref, h1_ref, c1_ref, y1_ref, disb_ref, y2_ref, w_ref, b_ref,
             out_ref):
        h1 = h1_ref[...]
        disb = disb_ref[...]
        tx1 = -disb * (y1_ref[0] + y1_ref[1])
        tx2 = -2.0 * disb * (y2_ref[0] + y2_ref[1]) - h1
        a = jnp.concatenate([x_ref[0], h1, tx1, tx2], axis=1)
        z = jnp.dot(a.astype(jnp.bfloat16),
                    w_ref[...].astype(jnp.bfloat16),
                    preferred_element_type=jnp.float32) + b_ref[...]
        zi, zf, zc, zo = _gates(z, D)
        c2 = zf * c1_ref[...] + zi * zc
        h2 = zo * jnp.tanh(c2)
        out_ref[...] = jnp.maximum(h2, 0.0)

    return pl.pallas_call(
        body,
        grid=(N // RB,),
        in_specs=[
            pl.BlockSpec((1, RB, D), lambda i: (1, i, 0)),
            pl.BlockSpec((RB, D), lambda i: (i, 0)),
            pl.BlockSpec((RB, D), lambda i: (i, 0)),
            pl.BlockSpec((NC, RB, D), lambda i: (0, i, 0)),
            pl.BlockSpec((RB, D), lambda i: (i, 0)),
            pl.BlockSpec((NC, RB, D), lambda i: (0, i, 0)),
            pl.BlockSpec((4 * D, 4 * D), lambda i: (0, 0)),
            pl.BlockSpec((1, 4 * D), lambda i: (0, 0)),
        ],
        out_specs=pl.BlockSpec((RB, D), lambda i: (i, 0)),
        out_shape=jax.ShapeDtypeStruct((N, D), jnp.float32),
    )


def kernel(edge_index_list, node_feats_list, edge_feats_list, nodes_mask_list,
           W_i, W_f, W_c, W_o, b_i, b_f, b_c, b_o,
           conv_i_W, conv_i_b, conv_f_W, conv_f_b,
           conv_c_W, conv_c_b, conv_o_W, conv_o_b):
    del nodes_mask_list
    Tn, N, D = node_feats_list.shape
    E = edge_index_list.shape[2]
    assert Tn == 2

    Wx = jnp.concatenate([W_i, W_f, W_c, W_o], axis=1)
    Wk = [jnp.concatenate([conv_i_W[k], conv_f_W[k],
                           conv_c_W[k], conv_o_W[k]], axis=1)
          for k in range(3)]
    Wbig = jnp.concatenate([Wx, Wk[0], Wk[1], Wk[2]], axis=0)
    bias = (jnp.concatenate([b_i, b_f, b_c, b_o], axis=1)
            + jnp.concatenate([conv_i_b, conv_f_b,
                               conv_c_b, conv_o_b])[None, :])

    RB = 1000
    deg_p = _make_deg(E, N, RB)(edge_index_list, edge_feats_list)
    H1, C1, U1, disb = _make_dense0(N, D, RB)(
        node_feats_list, Wx, bias, deg_p)
    Y1 = _make_prop(E, N, D)(U1, edge_index_list, edge_feats_list)
    U2 = _make_mid(N, D, RB)(Y1, disb)
    Y2 = _make_prop(E, N, D)(U2, edge_index_list, edge_feats_list)
    out = _make_final(N, D, RB)(
        node_feats_list, H1, C1, Y1, disb, Y2, Wbig, bias)
    return out


# speedup vs baseline: 2.2804x; 1.0025x over previous
import functools

import jax
import jax.numpy as jnp
from jax import lax
from jax.experimental import pallas as pl
from jax.experimental.pallas import tpu as pltpu
from jax.experimental.pallas import tpu_sc as plsc

NC = 2
NS = 16
NW = NC * NS


def _sc_mesh():
    return plsc.VectorSubcoreMesh(
        core_axis_name="c", subcore_axis_name="s",
        num_cores=NC, num_subcores=NS)


def _make_deg(E, N, RB):
    EPT = E // NW
    assert EPT % 16 == 0 and N % 16 == 0 and N % RB == 0

    @functools.partial(
        pl.kernel, mesh=_sc_mesh(),
        out_type=jax.ShapeDtypeStruct((N // RB, NW, RB), jnp.float32),
        compiler_params=pltpu.CompilerParams(
            needs_layout_passes=False, use_tc_tiling_on_sc=False),
        scratch_types=[
            pltpu.VMEM((N,), jnp.float32),
            pltpu.VMEM((EPT,), jnp.int32),
            pltpu.VMEM((EPT,), jnp.float32),
        ],
    )
    def deg_kernel(ei_hbm, ef_hbm, out_hbm, acc_v, src_all, ew_all):
        cid = lax.axis_index("c")
        sid = lax.axis_index("s")
        wid = sid * NC + cid
        base = wid * EPT
        pltpu.sync_copy(ei_hbm.at[1, 0, pl.ds(base, EPT)], src_all)
        pltpu.sync_copy(ef_hbm.at[1, pl.ds(base, EPT)], ew_all)
        zero = jnp.zeros((16,), jnp.float32)

        def zbody(i, carry):
            acc_v[pl.ds(i * 16, 16)] = zero
            return carry
        lax.fori_loop(0, N // 16, zbody, 0)

        def grp(g, c2):
            idx = src_all[pl.ds(g * 16, 16)]
            w = ew_all[pl.ds(g * 16, 16)]
            plsc.addupdate_scatter(acc_v, [idx], w)
            return c2
        lax.fori_loop(0, EPT // 16, grp, 0)

        for q in range(N // RB):
            pltpu.sync_copy(acc_v.at[pl.ds(q * RB, RB)], out_hbm.at[q, wid])

    return deg_kernel


def _make_prop(E, N, D):
    EPT = E // NW
    B = 40
    NB = EPT // B
    U = 5
    S = 3
    G = 2
    NPEEL = NB % U
    RPS = N // NS
    ZQ, ZREM = divmod(RPS, B)
    assert EPT % B == 0 and B % 8 == 0 and D % 16 == 0

    @functools.partial(
        pl.kernel, mesh=_sc_mesh(),
        out_type=jax.ShapeDtypeStruct((NC, N, D), jnp.float32),
        compiler_params=pltpu.CompilerParams(
            needs_layout_passes=False, use_tc_tiling_on_sc=False),
        scratch_types=[
            pltpu.VMEM_SHARED((N, D), jnp.float32),
            pltpu.VMEM((U, B, D), jnp.float32),
            pltpu.VMEM((U, B), jnp.int32),
            pltpu.VMEM((U, B), jnp.int32),
            pltpu.VMEM((U, B), jnp.float32),
            pltpu.SemaphoreType.DMA((U,)),
            pltpu.SemaphoreType.DMA((U,)),
            pltpu.SemaphoreType.DMA((U,)),
            pltpu.SemaphoreType.DMA((U,)),
            pltpu.SemaphoreType.DMA((U,)),
        ],
    )
    def prop_kernel(u_hbm, ei_hbm, ef_hbm, out_hbm,
                    acc_sh, rows, sidx, didx, ewr,
                    gsem, xsem, dsem, wsem, ssem):
        cid = lax.axis_index("c")
        sid = lax.axis_index("s")
        wid = sid * NC + cid
        base = wid * EPT

        zero = jnp.zeros((16,), jnp.float32)

        def zrow(i, carry):
            for c in range(D // 16):
                rows[0, i, pl.ds(c * 16, 16)] = zero
            return carry
        lax.fori_loop(0, B, zrow, 0)
        for q in range(ZQ):
            pltpu.sync_copy(rows.at[0],
                            acc_sh.at[pl.ds(sid * RPS + q * B, B)])
        if ZREM:
            pltpu.sync_copy(rows.at[0, pl.ds(0, ZREM)],
                            acc_sh.at[pl.ds(sid * RPS + ZQ * B, ZREM)])
        plsc.subcore_barrier()

        def issue_small(blk, slot):
            pltpu.async_copy(ei_hbm.at[1, 0, pl.ds(base + blk * B, B)],
                             sidx.at[slot], xsem.at[slot])
            pltpu.async_copy(ei_hbm.at[1, 1, pl.ds(base + blk * B, B)],
                             didx.at[slot], dsem.at[slot])
            pltpu.async_copy(ef_hbm.at[1, pl.ds(base + blk * B, B)],
                             ewr.at[slot], wsem.at[slot])

        def issue_gather(blk, slot):
            pltpu.make_async_copy(ei_hbm.at[1, 0, pl.ds(base + blk * B, B)],
                                  sidx.at[slot], xsem.at[slot]).wait()
            pltpu.async_copy(u_hbm.at[sidx.at[slot]],
                             rows.at[slot], gsem.at[slot])

        def wait_gather(blk, slot):
            pltpu.make_async_copy(u_hbm.at[sidx.at[slot]],
                                  rows.at[slot], gsem.at[slot]).wait()
            pltpu.make_async_copy(ef_hbm.at[1, pl.ds(base + blk * B, B)],
                                  ewr.at[slot], wsem.at[slot]).wait()

        def wait_scatter(slot):
            pltpu.make_async_copy(rows.at[slot], acc_sh.at[didx.at[slot]],
                                  ssem.at[slot]).wait()

        for a in range(S):
            issue_small(a, a)
        for a in range(G):
            issue_gather(a, a)

        def scale_16(a, w16, row0):
            for j in range(16):
                w = w16[j]
                row = row0 + j
                for c in range(D // 16):
                    rows[a, row, pl.ds(c * 16, 16)] = (
                        rows[a, row, pl.ds(c * 16, 16)] * w)

        def step(b, a):
            slot_s = (a + S) % U
            slot_g = (a + G) % U

            @pl.when((b >= U - S) & (b + S < NB))
            def _():
                wait_scatter(slot_s)

            @pl.when(b + S < NB)
            def _():
                issue_small(b + S, slot_s)

            @pl.when(b + G < NB)
            def _():
                issue_gather(b + G, slot_g)

            wait_gather(b, a)

            def scale(g, c2):
                scale_16(a, ewr[a, pl.ds(g * 16, 16)], g * 16)
                return c2
            lax.fori_loop(0, B // 16, scale, 0)
            if B % 16:
                rem = B % 16
                w16 = ewr[a, pl.ds(B - 16, 16)]
                for j in range(16 - rem, 16):
                    w = w16[j]
                    row = B - 16 + j
                    for c in range(D // 16):
                        rows[a, row, pl.ds(c * 16, 16)] = (
                            rows[a, row, pl.ds(c * 16, 16)] * w)

            pltpu.make_async_copy(ei_hbm.at[1, 1, pl.ds(base + b * B, B)],
                                  didx.at[a], dsem.at[a]).wait()
            pltpu.async_copy(rows.at[a], acc_sh.at[didx.at[a]],
                             ssem.at[a], add=True)

        def outer(ob, carry):
            for a in range(U):
                step(ob * U + a, a)
            return carry
        lax.fori_loop(0, NB // U, outer, 0)
        for p in range(NB - NPEEL, NB):
            step(p, p % U)

        for x in range(NB - U, NB):
            wait_scatter(x % U)

        plsc.subcore_barrier()
        pltpu.sync_copy(acc_sh.at[pl.ds(sid * RPS, RPS)],
                        out_hbm.at[cid, pl.ds(sid * RPS, RPS)])

    return prop_kernel


def _gates(z, D):
    zi = jax.nn.sigmoid(z[:, :D])
    zf = jax.nn.sigmoid(z[:, D:2 * D])
    zc = jnp.tanh(z[:, 2 * D:3 * D])
    zo = jax.nn.sigmoid(z[:, 3 * D:])
    return zi, zf, zc, zo


def _make_dense0(N, D, RB):
    def body(x_ref, w_ref, b_ref, h1_ref, c1_ref):
        z = jnp.dot(x_ref[0].astype(jnp.bfloat16),
                    w_ref[...].astype(jnp.bfloat16),
                    preferred_element_type=jnp.float32) + b_ref[...]
        zi, zf, zc, zo = _gates(z, D)
        c1 = zi * zc
        c1_ref[...] = c1
        h1_ref[...] = zo * jnp.tanh(c1)

    return pl.pallas_call(
        body,
        grid=(N // RB,),
        in_specs=[
            pl.BlockSpec((1, RB, D), lambda i: (0, i, 0)),
            pl.BlockSpec((D, 4 * D), lambda i: (0, 0)),
            pl.BlockSpec((1, 4 * D), lambda i: (0, 0)),
        ],
        out_specs=[
            pl.BlockSpec((RB, D), lambda i: (i, 0)),
            pl.BlockSpec((RB, D), lambda i: (i, 0)),
        ],
        out_shape=[
            jax.ShapeDtypeStruct((N, D), jnp.float32),
            jax.ShapeDtypeStruct((N, D), jnp.float32),
        ],
    )


def _make_scale0(N, D, RB):
    def body(degp_ref, h1_ref, u1_ref, disb_ref):
        deg = jnp.sum(degp_ref[0], axis=0)
        deg_safe = jnp.where(deg > 0, deg, 1.0)
        dis = jnp.where(deg > 0, lax.rsqrt(deg_safe), 0.0)
        disb = jnp.broadcast_to(dis[:, None], (RB, D))
        disb_ref[...] = disb
        u1_ref[...] = disb * h1_ref[...]

    return pl.pallas_call(
        body,
        grid=(N // RB,),
        in_specs=[
            pl.BlockSpec((1, NW, RB), lambda i: (i, 0, 0)),
            pl.BlockSpec((RB, D), lambda i: (i, 0)),
        ],
        out_specs=[
            pl.BlockSpec((RB, D), lambda i: (i, 0)),
            pl.BlockSpec((RB, D), lambda i: (i, 0)),
        ],
        out_shape=[
            jax.ShapeDtypeStruct((N, D), jnp.float32),
            jax.ShapeDtypeStruct((N, D), jnp.float32),
        ],
    )


def _make_mid(N, D, RB):
    def body(y_ref, disb_ref, u2_ref):
        disb = disb_ref[...]
        u2_ref[...] = -(disb * disb) * (y_ref[0] + y_ref[1])

    return pl.pallas_call(
        body,
        grid=(N // RB,),
        in_specs=[
            pl.BlockSpec((NC, RB, D), lambda i: (0, i, 0)),
            pl.BlockSpec((RB, D), lambda i: (i, 0)),
        ],
        out_specs=pl.BlockSpec((RB, D), lambda i: (i, 0)),
        out_shape=jax.ShapeDtypeStruct((N, D), jnp.float32),
    )


def _make_final(N, D, RB):
    def body(x_ref, h1_ref, c1_ref, y1_ref, disb_ref, y2_ref, w_ref, b_ref,
             out_ref):
        h1 = h1_ref[...]
        disb = disb_ref[...]
        tx1 = -disb * (y1_ref[0] + y1_ref[1])
        tx2 = -2.0 * disb * (y2_ref[0] + y2_ref[1]) - h1
        a = jnp.concatenate([x_ref[0], h1, tx1, tx2], axis=1)
        z = jnp.dot(a.astype(jnp.bfloat16),
                    w_ref[...].astype(jnp.bfloat16),
                    preferred_element_type=jnp.float32) + b_ref[...]
        zi, zf, zc, zo = _gates(z, D)
        c2 = zf * c1_ref[...] + zi * zc
        h2 = zo * jnp.tanh(c2)
        out_ref[...] = jnp.maximum(h2, 0.0)

    return pl.pallas_call(
        body,
        grid=(N // RB,),
        in_specs=[
            pl.BlockSpec((1, RB, D), lambda i: (1, i, 0)),
            pl.BlockSpec((RB, D), lambda i: (i, 0)),
            pl.BlockSpec((RB, D), lambda i: (i, 0)),
            pl.BlockSpec((NC, RB, D), lambda i: (0, i, 0)),
            pl.BlockSpec((RB, D), lambda i: (i, 0)),
            pl.BlockSpec((NC, RB, D), lambda i: (0, i, 0)),
            pl.BlockSpec((4 * D, 4 * D), lambda i: (0, 0)),
            pl.BlockSpec((1, 4 * D), lambda i: (0, 0)),
        ],
        out_specs=pl.BlockSpec((RB, D), lambda i: (i, 0)),
        out_shape=jax.ShapeDtypeStruct((N, D), jnp.float32),
    )


def kernel(edge_index_list, node_feats_list, edge_feats_list, nodes_mask_list,
           W_i, W_f, W_c, W_o, b_i, b_f, b_c, b_o,
           conv_i_W, conv_i_b, conv_f_W, conv_f_b,
           conv_c_W, conv_c_b, conv_o_W, conv_o_b):
    del nodes_mask_list
    Tn, N, D = node_feats_list.shape
    E = edge_index_list.shape[2]
    assert Tn == 2

    Wx = jnp.concatenate([W_i, W_f, W_c, W_o], axis=1)
    Wk = [jnp.concatenate([conv_i_W[k], conv_f_W[k],
                           conv_c_W[k], conv_o_W[k]], axis=1)
          for k in range(3)]
    Wbig = jnp.concatenate([Wx, Wk[0], Wk[1], Wk[2]], axis=0)
    bias = (jnp.concatenate([b_i, b_f, b_c, b_o], axis=1)
            + jnp.concatenate([conv_i_b, conv_f_b,
                               conv_c_b, conv_o_b])[None, :])

    RB = 1000
    deg_p = _make_deg(E, N, RB)(edge_index_list, edge_feats_list)
    H1, C1 = _make_dense0(N, D, RB)(node_feats_list, Wx, bias)
    U1, disb = _make_scale0(N, D, RB)(deg_p, H1)
    Y1 = _make_prop(E, N, D)(U1, edge_index_list, edge_feats_list)
    U2 = _make_mid(N, D, RB)(Y1, disb)
    Y2 = _make_prop(E, N, D)(U2, edge_index_list, edge_feats_list)
    out = _make_final(N, D, RB)(
        node_feats_list, H1, C1, Y1, disb, Y2, Wbig, bias)
    return out


# speedup vs baseline: 2.3371x; 1.0248x over previous
import functools

import jax
import jax.numpy as jnp
from jax import lax
from jax.experimental import pallas as pl
from jax.experimental.pallas import tpu as pltpu
from jax.experimental.pallas import tpu_sc as plsc

NC = 2
NS = 16
NW = NC * NS


def _sc_mesh():
    return plsc.VectorSubcoreMesh(
        core_axis_name="c", subcore_axis_name="s",
        num_cores=NC, num_subcores=NS)


def _make_deg(E, N, RB):
    EPT = E // NW
    assert EPT % 16 == 0 and N % 16 == 0 and N % RB == 0

    @functools.partial(
        pl.kernel, mesh=_sc_mesh(),
        out_type=jax.ShapeDtypeStruct((N // RB, NW, RB), jnp.float32),
        compiler_params=pltpu.CompilerParams(
            needs_layout_passes=False, use_tc_tiling_on_sc=False),
        scratch_types=[
            pltpu.VMEM((N,), jnp.float32),
            pltpu.VMEM((EPT,), jnp.int32),
            pltpu.VMEM((EPT,), jnp.float32),
        ],
    )
    def deg_kernel(ei_hbm, ef_hbm, out_hbm, acc_v, src_all, ew_all):
        cid = lax.axis_index("c")
        sid = lax.axis_index("s")
        wid = sid * NC + cid
        base = wid * EPT
        pltpu.sync_copy(ei_hbm.at[1, 0, pl.ds(base, EPT)], src_all)
        pltpu.sync_copy(ef_hbm.at[1, pl.ds(base, EPT)], ew_all)
        zero = jnp.zeros((16,), jnp.float32)

        def zbody(i, carry):
            acc_v[pl.ds(i * 16, 16)] = zero
            return carry
        lax.fori_loop(0, N // 16, zbody, 0)

        def grp(g, c2):
            idx = src_all[pl.ds(g * 16, 16)]
            w = ew_all[pl.ds(g * 16, 16)]
            plsc.addupdate_scatter(acc_v, [idx], w)
            return c2
        lax.fori_loop(0, EPT // 16, grp, 0)

        for q in range(N // RB):
            pltpu.sync_copy(acc_v.at[pl.ds(q * RB, RB)], out_hbm.at[q, wid])

    return deg_kernel


def _make_prop(E, N, D):
    EPT = E // NW
    B = 40
    NB = EPT // B
    U = 5
    S = 3
    G = 2
    NPEEL = NB % U
    RPS = N // NS
    ZQ, ZREM = divmod(RPS, B)
    assert EPT % B == 0 and B % 8 == 0 and D % 16 == 0

    @functools.partial(
        pl.kernel, mesh=_sc_mesh(),
        out_type=jax.ShapeDtypeStruct((NC, N, D), jnp.float32),
        compiler_params=pltpu.CompilerParams(
            needs_layout_passes=False, use_tc_tiling_on_sc=False),
        scratch_types=[
            pltpu.VMEM_SHARED((N, D), jnp.float32),
            pltpu.VMEM((U, B, D), jnp.float32),
            pltpu.VMEM((U, B), jnp.int32),
            pltpu.VMEM((U, B), jnp.int32),
            pltpu.VMEM((U, B), jnp.float32),
            pltpu.SemaphoreType.DMA((U,)),
            pltpu.SemaphoreType.DMA((U,)),
            pltpu.SemaphoreType.DMA((U,)),
            pltpu.SemaphoreType.DMA((U,)),
            pltpu.SemaphoreType.DMA((U,)),
        ],
    )
    def prop_kernel(u_hbm, ei_hbm, ef_hbm, out_hbm,
                    acc_sh, rows, sidx, didx, ewr,
                    gsem, xsem, dsem, wsem, ssem):
        cid = lax.axis_index("c")
        sid = lax.axis_index("s")
        wid = sid * NC + cid
        base = wid * EPT

        zero = jnp.zeros((16,), jnp.float32)

        def zrow(i, carry):
            for c in range(D // 16):
                rows[0, i, pl.ds(c * 16, 16)] = zero
            return carry
        lax.fori_loop(0, B, zrow, 0)
        for q in range(ZQ):
            pltpu.sync_copy(rows.at[0],
                            acc_sh.at[pl.ds(sid * RPS + q * B, B)])
        if ZREM:
            pltpu.sync_copy(rows.at[0, pl.ds(0, ZREM)],
                            acc_sh.at[pl.ds(sid * RPS + ZQ * B, ZREM)])
        plsc.subcore_barrier()

        def issue_small(blk, slot):
            pltpu.async_copy(ei_hbm.at[1, 0, pl.ds(base + blk * B, B)],
                             sidx.at[slot], xsem.at[slot])
            pltpu.async_copy(ei_hbm.at[1, 1, pl.ds(base + blk * B, B)],
                             didx.at[slot], dsem.at[slot])
            pltpu.async_copy(ef_hbm.at[1, pl.ds(base + blk * B, B)],
                             ewr.at[slot], wsem.at[slot])

        def issue_gather(blk, slot):
            pltpu.make_async_copy(ei_hbm.at[1, 0, pl.ds(base + blk * B, B)],
                                  sidx.at[slot], xsem.at[slot]).wait()
            pltpu.async_copy(u_hbm.at[sidx.at[slot]],
                             rows.at[slot], gsem.at[slot])

        def wait_gather(blk, slot):
            pltpu.make_async_copy(u_hbm.at[sidx.at[slot]],
                                  rows.at[slot], gsem.at[slot]).wait()
            pltpu.make_async_copy(ef_hbm.at[1, pl.ds(base + blk * B, B)],
                                  ewr.at[slot], wsem.at[slot]).wait()

        def wait_scatter(slot):
            pltpu.make_async_copy(rows.at[slot], acc_sh.at[didx.at[slot]],
                                  ssem.at[slot]).wait()

        for a in range(S):
            issue_small(a, a)
        for a in range(G):
            issue_gather(a, a)

        def scale_16(a, w16, row0):
            for j in range(16):
                w = w16[j]
                row = row0 + j
                for c in range(D // 16):
                    rows[a, row, pl.ds(c * 16, 16)] = (
                        rows[a, row, pl.ds(c * 16, 16)] * w)

        def step(b, a):
            slot_s = (a + S) % U
            slot_g = (a + G) % U

            @pl.when((b >= U - S) & (b + S < NB))
            def _():
                wait_scatter(slot_s)

            @pl.when(b + S < NB)
            def _():
                issue_small(b + S, slot_s)

            @pl.when(b + G < NB)
            def _():
                issue_gather(b + G, slot_g)

            wait_gather(b, a)

            def scale(g, c2):
                scale_16(a, ewr[a, pl.ds(g * 16, 16)], g * 16)
                return c2
            lax.fori_loop(0, B // 16, scale, 0)
            if B % 16:
                rem = B % 16
                w16 = ewr[a, pl.ds(B - 16, 16)]
                for j in range(16 - rem, 16):
                    w = w16[j]
                    row = B - 16 + j
                    for c in range(D // 16):
                        rows[a, row, pl.ds(c * 16, 16)] = (
                            rows[a, row, pl.ds(c * 16, 16)] * w)

            pltpu.make_async_copy(ei_hbm.at[1, 1, pl.ds(base + b * B, B)],
                                  didx.at[a], dsem.at[a]).wait()
            pltpu.async_copy(rows.at[a], acc_sh.at[didx.at[a]],
                             ssem.at[a], add=True)

        def outer(ob, carry):
            for a in range(U):
                step(ob * U + a, a)
            return carry
        lax.fori_loop(0, NB // U, outer, 0)
        for p in range(NB - NPEEL, NB):
            step(p, p % U)

        for x in range(NB - U, NB):
            wait_scatter(x % U)

        plsc.subcore_barrier()
        pltpu.sync_copy(acc_sh.at[pl.ds(sid * RPS, RPS)],
                        out_hbm.at[cid, pl.ds(sid * RPS, RPS)])

    return prop_kernel


def _gates(z, D):
    zi = jax.nn.sigmoid(z[:, :D])
    zf = jax.nn.sigmoid(z[:, D:2 * D])
    zc = jnp.tanh(z[:, 2 * D:3 * D])
    zo = jax.nn.sigmoid(z[:, 3 * D:])
    return zi, zf, zc, zo


def _make_dense0(N, D, RB):
    def body(x_ref, w_ref, b_ref, h1_ref, c1_ref):
        z = jnp.dot(x_ref[0].astype(jnp.bfloat16),
                    w_ref[...].astype(jnp.bfloat16),
                    preferred_element_type=jnp.float32) + b_ref[...]
        zi, zf, zc, zo = _gates(z, D)
        c1 = zi * zc
        c1_ref[...] = c1
        h1_ref[...] = zo * jnp.tanh(c1)

    return pl.pallas_call(
        body,
        grid=(N // RB,),
        in_specs=[
            pl.BlockSpec((1, RB, D), lambda i: (0, i, 0)),
            pl.BlockSpec((D, 4 * D), lambda i: (0, 0)),
            pl.BlockSpec((1, 4 * D), lambda i: (0, 0)),
        ],
        out_specs=[
            pl.BlockSpec((RB, D), lambda i: (i, 0)),
            pl.BlockSpec((RB, D), lambda i: (i, 0)),
        ],
        out_shape=[
            jax.ShapeDtypeStruct((N, D), jnp.float32),
            jax.ShapeDtypeStruct((N, D), jnp.float32),
        ],
    )


def _make_scale0(N, D, RB):
    def body(degp_ref, h1_ref, u1_ref, disb_ref):
        deg = jnp.sum(degp_ref[0], axis=0)
        deg_safe = jnp.where(deg > 0, deg, 1.0)
        dis = jnp.where(deg > 0, lax.rsqrt(deg_safe), 0.0)
        disb = jnp.broadcast_to(dis[:, None], (RB, D))
        disb_ref[...] = disb
        u1_ref[...] = disb * h1_ref[...]

    return pl.pallas_call(
        body,
        grid=(N // RB,),
        in_specs=[
            pl.BlockSpec((1, NW, RB), lambda i: (i, 0, 0)),
            pl.BlockSpec((RB, D), lambda i: (i, 0)),
        ],
        out_specs=[
            pl.BlockSpec((RB, D), lambda i: (i, 0)),
            pl.BlockSpec((RB, D), lambda i: (i, 0)),
        ],
        out_shape=[
            jax.ShapeDtypeStruct((N, D), jnp.float32),
            jax.ShapeDtypeStruct((N, D), jnp.float32),
        ],
    )


def _make_mid(N, D, RB):
    def body(y_ref, disb_ref, u2_ref):
        disb = disb_ref[...]
        u2_ref[...] = -(disb * disb) * (y_ref[0] + y_ref[1])

    return pl.pallas_call(
        body,
        grid=(N // RB,),
        in_specs=[
            pl.BlockSpec((NC, RB, D), lambda i: (0, i, 0)),
            pl.BlockSpec((RB, D), lambda i: (i, 0)),
        ],
        out_specs=pl.BlockSpec((RB, D), lambda i: (i, 0)),
        out_shape=jax.ShapeDtypeStruct((N, D), jnp.float32),
    )


def _make_final(N, D, RB):
    def body(x_ref, h1_ref, c1_ref, y1_ref, disb_ref, y2_ref, w_ref, b_ref,
             out_ref):
        h1 = h1_ref[...]
        disb = disb_ref[...]
        tx1 = -disb * (y1_ref[0] + y1_ref[1])
        tx2 = -2.0 * disb * (y2_ref[0] + y2_ref[1]) - h1
        a = jnp.concatenate([x_ref[0], h1, tx1, tx2], axis=1)
        z = jnp.dot(a.astype(jnp.bfloat16),
                    w_ref[...].astype(jnp.bfloat16),
                    preferred_element_type=jnp.float32) + b_ref[...]
        zi, zf, zc, zo = _gates(z, D)
        c2 = zf * c1_ref[...] + zi * zc
        h2 = zo * jnp.tanh(c2)
        out_ref[...] = jnp.maximum(h2, 0.0)

    return pl.pallas_call(
        body,
        grid=(N // RB,),
        in_specs=[
            pl.BlockSpec((1, RB, D), lambda i: (1, i, 0)),
            pl.BlockSpec((RB, D), lambda i: (i, 0)),
            pl.BlockSpec((RB, D), lambda i: (i, 0)),
            pl.BlockSpec((NC, RB, D), lambda i: (0, i, 0)),
            pl.BlockSpec((RB, D), lambda i: (i, 0)),
            pl.BlockSpec((NC, RB, D), lambda i: (0, i, 0)),
            pl.BlockSpec((4 * D, 4 * D), lambda i: (0, 0)),
            pl.BlockSpec((1, 4 * D), lambda i: (0, 0)),
        ],
        out_specs=pl.BlockSpec((RB, D), lambda i: (i, 0)),
        out_shape=jax.ShapeDtypeStruct((N, D), jnp.float32),
    )


def kernel(edge_index_list, node_feats_list, edge_feats_list, nodes_mask_list,
           W_i, W_f, W_c, W_o, b_i, b_f, b_c, b_o,
           conv_i_W, conv_i_b, conv_f_W, conv_f_b,
           conv_c_W, conv_c_b, conv_o_W, conv_o_b):
    del nodes_mask_list
    Tn, N, D = node_feats_list.shape
    E = edge_index_list.shape[2]
    assert Tn == 2

    Wx = jnp.concatenate([W_i, W_f, W_c, W_o], axis=1)
    Wk = [jnp.concatenate([conv_i_W[k], conv_f_W[k],
                           conv_c_W[k], conv_o_W[k]], axis=1)
          for k in range(3)]
    Wbig = jnp.concatenate([Wx, Wk[0], Wk[1], Wk[2]], axis=0)
    bias = (jnp.concatenate([b_i, b_f, b_c, b_o], axis=1)
            + jnp.concatenate([conv_i_b, conv_f_b,
                               conv_c_b, conv_o_b])[None, :])

    RB = 2000
    deg_p = _make_deg(E, N, RB)(edge_index_list, edge_feats_list)
    H1, C1 = _make_dense0(N, D, RB)(node_feats_list, Wx, bias)
    U1, disb = _make_scale0(N, D, RB)(deg_p, H1)
    Y1 = _make_prop(E, N, D)(U1, edge_index_list, edge_feats_list)
    U2 = _make_mid(N, D, RB)(Y1, disb)
    Y2 = _make_prop(E, N, D)(U2, edge_index_list, edge_feats_list)
    out = _make_final(N, D, RB)(
        node_feats_list, H1, C1, Y1, disb, Y2, Wbig, bias)
    return out
